# baseline (speedup 1.0000x reference)
import dataclasses
import functools

import jax
import jax.numpy as jnp
import numpy as np
from jax import lax
from jax.experimental import pallas as pl
from jax.experimental.pallas import tpu as pltpu
from jax.experimental.pallas import tpu_sc as plsc

BN_EPS = 1e-5
_BNC = 1.0 / (1.0 + BN_EPS) ** 0.5

SC_CORES = 2
SC_SUBCORES = 16
SC_LANES = 16
SC_WORKERS = SC_CORES * SC_SUBCORES


def _edge_q_body(ea_ref, w1_ref, b1_ref, w2_ref, b2_ref, we_ref, be_ref,
                 q0_ref, q1_ref, q2_ref, q3_ref):
    ea = ea_ref[...]
    t = jnp.maximum(
        jnp.dot(ea, w1_ref[...], preferred_element_type=jnp.float32)
        + b1_ref[...], 0.0)
    e = (jnp.dot(t, w2_ref[...], preferred_element_type=jnp.float32)
         + b2_ref[...])
    for l, q_ref in enumerate((q0_ref, q1_ref, q2_ref, q3_ref)):
        q_ref[...] = (
            jnp.dot(e, we_ref[l], preferred_element_type=jnp.float32)
            + be_ref[l]).astype(jnp.bfloat16)


def _edge_q(edge_attr, ee_W1, ee_b1, ee_W2, ee_b2, We, be, *, block_e):
    E, D = edge_attr.shape
    H = ee_W2.shape[1]
    nblk = E // block_e
    out_sds = jax.ShapeDtypeStruct((E, H), jnp.bfloat16)
    row_spec = pl.BlockSpec((block_e, H), lambda i: (i, 0))
    full = lambda s: pl.BlockSpec(s, lambda i: (0,) * len(s))
    return pl.pallas_call(
        _edge_q_body,
        grid=(nblk,),
        in_specs=[
            pl.BlockSpec((block_e, D), lambda i: (i, 0)),
            full(ee_W1.shape), full((1, H)), full(ee_W2.shape), full((1, H)),
            full(We.shape), full((4, 1, H)),
        ],
        out_specs=[row_spec] * 4,
        out_shape=[out_sds] * 4,
    )(edge_attr, ee_W1, ee_b1.reshape(1, H), ee_W2, ee_b2.reshape(1, H),
      We, be.reshape(4, 1, H))


def _pack_edge_pairs(a_bf16):
    E, H = a_bf16.shape
    return jax.lax.bitcast_convert_type(
        a_bf16.reshape(E // 2, H, 2), jnp.int32)


def _xprep_body(x_ref, pmat_ref, xp_ref):
    xp_ref[...] = jnp.dot(x_ref[...], pmat_ref[...],
                          preferred_element_type=jnp.float32)


def _x_prep(x, pmat, *, block_n):
    N, H = x.shape
    nblk = N // block_n
    full = lambda s: pl.BlockSpec(s, lambda i: (0,) * len(s))
    row_spec = pl.BlockSpec((block_n, H), lambda i: (i, 0))
    return pl.pallas_call(
        _xprep_body,
        grid=(nblk,),
        in_specs=[row_spec, full(pmat.shape)],
        out_specs=row_spec,
        out_shape=jax.ShapeDtypeStruct((N, H), jnp.float32),
    )(x, pmat)


def _sc_message_body(n_pad, n_edges, chunk, h_hbm, q_hbm, src_hbm, dst_hbm,
                     z_hbm, out_hbm, srcv, dstv, qv, rowsv, aggr,
                     sem_ld, sem_g, sem_sc):
    H = h_hbm.shape[1]
    c = lax.axis_index("c")
    s = lax.axis_index("s")
    wid = c * SC_SUBCORES + s
    per_w = n_edges // SC_WORKERS
    base = wid * per_w
    nchunks = per_w // chunk

    base2 = wid * (per_w // 2)
    chunk2 = chunk // 2

    def issue_loads(b2, d4, ch):
        off = base + ch * chunk
        pltpu.async_copy(src_hbm.at[pl.ds(off, chunk)], srcv.at[b2],
                         sem_ld.at[b2])
        pltpu.async_copy(dst_hbm.at[pl.ds(off, chunk)], dstv.at[d4],
                         sem_ld.at[b2])
        pltpu.async_copy(q_hbm.at[pl.ds(base2 + ch * chunk2, chunk2)],
                         qv.at[b2], sem_ld.at[b2])

    def wait_loads(b2, d4):
        pltpu.make_async_copy(src_hbm.at[pl.ds(0, chunk)], srcv.at[b2],
                              sem_ld.at[b2]).wait()
        pltpu.make_async_copy(dst_hbm.at[pl.ds(0, chunk)], dstv.at[d4],
                              sem_ld.at[b2]).wait()
        pltpu.make_async_copy(q_hbm.at[pl.ds(0, chunk2)], qv.at[b2],
                              sem_ld.at[b2]).wait()

    def issue_gather(b2):
        pltpu.async_copy(h_hbm.at[srcv.at[b2]], rowsv.at[b2], sem_g.at[b2])

    def wait_gather(b2):
        pltpu.make_async_copy(h_hbm.at[srcv.at[b2]], rowsv.at[b2],
                              sem_g.at[b2]).wait()

    MASK_HI = jnp.int32(-65536)

    def compute(b2):
        rows = rowsv.at[b2]
        qb = qv.at[b2]

        @pl.loop(0, chunk2)
        def _(j):
            for e in range(2):
                row = 2 * j + e
                for g in range(H // (2 * SC_LANES)):
                    qw = qb.at[pl.ds(j, 1),
                               pl.ds(e * (H // 2) + g * SC_LANES,
                                     SC_LANES)][...]
                    q_lo = lax.bitcast_convert_type(qw << 16, jnp.float32)
                    q_hi = lax.bitcast_convert_type(qw & MASK_HI,
                                                    jnp.float32)
                    p0 = 2 * g * SC_LANES
                    sl_lo = (pl.ds(row, 1), pl.ds(p0, SC_LANES))
                    sl_hi = (pl.ds(row, 1), pl.ds(p0 + SC_LANES, SC_LANES))
                    rows.at[*sl_lo][...] = jnp.maximum(
                        rows.at[*sl_lo][...] + q_lo, 0.0)
                    rows.at[*sl_hi][...] = jnp.maximum(
                        rows.at[*sl_hi][...] + q_hi, 0.0)

    def issue_scatter(b2, d4):
        pltpu.async_copy(rowsv.at[b2], aggr.at[dstv.at[d4]], sem_sc.at[b2],
                         add=True)

    def wait_scatter(b2, d4):
        pltpu.make_async_copy(rowsv.at[b2], aggr.at[dstv.at[d4]],
                              sem_sc.at[b2]).wait()

    issue_loads(0, 0, 0)
    issue_loads(1, 1, 1)
    rows_per_sub = n_pad // SC_SUBCORES
    r0 = s * rows_per_sub
    pltpu.sync_copy(z_hbm.at[pl.ds(r0, rows_per_sub)],
                    aggr.at[pl.ds(r0, rows_per_sub)])
    plsc.subcore_barrier()

    wait_loads(0, 0)
    issue_gather(0)

    nmain = (nchunks - 1) // 4
    assert nmain * 4 == nchunks - 1, "nchunks must be 1 mod 4"

    @pl.loop(0, nmain)
    def _(i):
        for k in range(4):
            ch = i * 4 + k
            b2, o2 = k % 2, (k + 1) % 2
            wait_gather(b2)
            if k == 0:
                @pl.when(i >= 1)
                def _():
                    wait_scatter(o2, 3)
            else:
                wait_scatter(o2, k - 1)
            wait_loads(o2, (k + 1) % 4)
            issue_gather(o2)
            compute(b2)
            issue_scatter(b2, k)
            if k < 3:
                issue_loads(b2, (k + 2) % 4, ch + 2)
            else:
                @pl.when(ch + 2 < nchunks)
                def _():
                    issue_loads(b2, (k + 2) % 4, ch + 2)

    wait_gather(0)
    wait_scatter(1, 3)
    compute(0)
    issue_scatter(0, 0)
    wait_scatter(0, 0)

    plsc.subcore_barrier()
    pltpu.sync_copy(aggr.at[pl.ds(r0, rows_per_sub)],
                    out_hbm.at[c].at[pl.ds(r0, rows_per_sub)])


def _sc_message(h, q_w, src, dst, zeros_pad, *, chunk):
    N, H = h.shape
    E = src.shape[0]
    n_pad = zeros_pad.shape[0]
    mesh = plsc.VectorSubcoreMesh(core_axis_name="c", subcore_axis_name="s")
    body = functools.partial(_sc_message_body, n_pad, E, chunk)
    cp = pltpu.CompilerParams()
    k = pl.kernel(
        body,
        out_type=jax.ShapeDtypeStruct((SC_CORES, n_pad, H), jnp.float32),
        mesh=mesh,
        compiler_params=cp,
        scratch_types=[
            pltpu.VMEM((2, chunk), jnp.int32),
            pltpu.VMEM((4, chunk), jnp.int32),
            pltpu.VMEM((2, chunk // 2, H), jnp.int32),
            pltpu.VMEM((2, chunk, H), jnp.float32),
            pltpu.VMEM_SHARED((n_pad, H), jnp.float32),
            pltpu.SemaphoreType.DMA((2,)),
            pltpu.SemaphoreType.DMA((2,)),
            pltpu.SemaphoreType.DMA((2,)),
        ],
    )
    return k(h, q_w, src, dst, zeros_pad)


def _padded_zeros(N, H):
    quantum = SC_SUBCORES * 8
    n_pad = ((N + quantum - 1) // quantum) * quantum
    return jnp.zeros((n_pad, H), jnp.float32)


def _node_mlp_body(h_ref, p_ref, w1_ref, b1_ref, w2_ref, b2_ref, gb_ref,
                   o_ref):
    z = h_ref[...] + p_ref[0] + p_ref[1]
    z = jnp.maximum(
        jnp.dot(z, w1_ref[...], preferred_element_type=jnp.float32)
        + b1_ref[...], 0.0)
    z = (jnp.dot(z, w2_ref[...], preferred_element_type=jnp.float32)
         + b2_ref[...])
    z = jnp.maximum(z, 0.0)
    o_ref[...] = z * (_BNC * gb_ref[0]) + gb_ref[1]


def _node_mlp(h, partials, W1p, b1, W2p, b2p, gp, betap, *, block_n):
    N, H = h.shape
    nblk = N // block_n
    full = lambda s: pl.BlockSpec(s, lambda i: (0,) * len(s))
    row_spec = pl.BlockSpec((block_n, H), lambda i: (i, 0))
    return pl.pallas_call(
        _node_mlp_body,
        grid=(nblk,),
        in_specs=[
            row_spec,
            pl.BlockSpec((SC_CORES, block_n, H), lambda i: (0, i, 0)),
            full(W1p.shape), full((1, H)), full(W2p.shape), full((1, H)),
            full((2, 1, H)),
        ],
        out_specs=row_spec,
        out_shape=jax.ShapeDtypeStruct((N, H), jnp.float32),
    )(h, partials, W1p, b1.reshape(1, H), W2p, b2p.reshape(1, H),
      jnp.stack([gp, betap]).reshape(2, 1, H))


def _pool_head_body(h_ref, b3_ref, lat_ref, lw1_ref, lb1_ref, lgb_ref,
                    lw2_ref, lb2_ref, fw1_ref, fb1_ref, fgb_ref, fw2_ref,
                    fb2_ref, o_ref, sum_ref, cnt_ref, *, n_groups):
    i = pl.program_id(0)
    nblk = pl.num_programs(0)

    @pl.when(i == 0)
    def _():
        sum_ref[...] = jnp.zeros_like(sum_ref)
        cnt_ref[...] = jnp.zeros_like(cnt_ref)

    b = b3_ref[0]
    gids = lax.broadcasted_iota(jnp.int32, (n_groups, b.shape[1]), 0)
    onehot = (b == gids).astype(jnp.float32)
    sum_ref[...] += jnp.dot(onehot, h_ref[...],
                            preferred_element_type=jnp.float32)
    cnt_ref[...] += jnp.sum(onehot, axis=1, keepdims=True)

    @pl.when(i == nblk - 1)
    def _():
        pooled = sum_ref[...] / jnp.maximum(cnt_ref[...], 1.0)
        lf = jnp.maximum(
            jnp.dot(lat_ref[...], lw1_ref[...],
                    preferred_element_type=jnp.float32) + lb1_ref[...], 0.0)
        lf = lf * (_BNC * lgb_ref[0]) + lgb_ref[1]
        lf = (jnp.dot(lf, lw2_ref[...], preferred_element_type=jnp.float32)
              + lb2_ref[...])
        H = pooled.shape[1]
        y = (jnp.dot(pooled, fw1_ref[pl.ds(0, H)],
                     preferred_element_type=jnp.float32)
             + jnp.dot(lf, fw1_ref[pl.ds(H, H)],
                       preferred_element_type=jnp.float32)
             + fb1_ref[...])
        y = jnp.maximum(y, 0.0)
        y = y * (_BNC * fgb_ref[0]) + fgb_ref[1]
        o_ref[...] = (jnp.dot(y, fw2_ref[...],
                              preferred_element_type=jnp.float32)
                      + fb2_ref[...])


def _pool_head(h, batch, lattice, p, *, block_n):
    N, H = h.shape
    G = lattice.shape[0]
    NC_OUT = p['f_W2'].shape[1]
    nblk = N // block_n
    batch3 = batch.reshape(nblk, 1, block_n)
    full = lambda s: pl.BlockSpec(s, lambda i: (0,) * len(s))
    body = functools.partial(_pool_head_body, n_groups=G)
    return pl.pallas_call(
        body,
        grid=(nblk,),
        in_specs=[
            pl.BlockSpec((block_n, H), lambda i: (i, 0)),
            pl.BlockSpec((1, 1, block_n), lambda i: (i, 0, 0)),
            full(lattice.shape), full(p['lat_W1'].shape), full((1, H)),
            full((2, 1, H)), full(p['lat_W2'].shape), full((1, H)),
            full(p['f_W1'].shape), full((1, H)), full((2, 1, H)),
            full(p['f_W2'].shape), full((1, NC_OUT)),
        ],
        out_specs=full((G, NC_OUT)),
        out_shape=jax.ShapeDtypeStruct((G, NC_OUT), jnp.float32),
        scratch_shapes=[
            pltpu.VMEM((G, H), jnp.float32),
            pltpu.VMEM((G, 1), jnp.float32),
        ],
    )(h, batch3, lattice, p['lat_W1'], p['lat_b1'].reshape(1, H),
      jnp.stack([p['lat_g'], p['lat_beta']]).reshape(2, 1, H),
      p['lat_W2'], p['lat_b2'].reshape(1, H),
      p['f_W1'], p['f_b1'].reshape(1, H),
      jnp.stack([p['f_g'], p['f_beta']]).reshape(2, 1, H),
      p['f_W2'], p['f_b2'].reshape(1, NC_OUT))


def _interleave_perm(H):
    perm = np.zeros(H, np.int64)
    for pos in range(H):
        g, o = divmod(pos, 2 * SC_LANES)
        f = 2 * o if o < SC_LANES else 2 * (o - SC_LANES) + 1
        perm[pos] = 2 * SC_LANES * g + f
    return perm


def kernel(x, edge_attr, lattice, params, edge_index, batch):
    p = params
    N, H = x.shape
    src = edge_index[0]
    dst = edge_index[1]

    perm = _interleave_perm(H)
    pmat = np.zeros((H, H), np.float32)
    pmat[perm, np.arange(H)] = 1.0
    pmat = jnp.asarray(pmat)

    We = jnp.stack([lp['We'] for lp in p['gnn']])
    be = jnp.stack([lp['be'] for lp in p['gnn']])

    qs = _edge_q(edge_attr, p['ee_W1'], p['ee_b1'], p['ee_W2'], p['ee_b2'],
                 We, be, block_e=1000)
    qws = [_pack_edge_pairs(q) for q in qs]

    zeros_pad = _padded_zeros(N, H)
    h = _x_prep(x, pmat, block_n=1000)
    for l, lp in enumerate(p['gnn']):
        partials = _sc_message(h, qws[l], src, dst, zeros_pad, chunk=80)
        h = _node_mlp(h, partials, lp['W1'][perm, :], lp['b1'],
                      lp['W2'][:, perm], lp['b2'][perm], lp['g'][perm],
                      lp['beta'][perm], block_n=1000)

    fW1p = jnp.concatenate([p['f_W1'][:H][perm, :], p['f_W1'][H:]], axis=0)
    p_head = dict(p)
    p_head['f_W1'] = fW1p
    return _pool_head(h, batch, lattice, p_head, block_n=1000)


# speedup vs baseline: 45.0378x; 45.0378x over previous
import dataclasses
import functools

import jax
import jax.numpy as jnp
import numpy as np
from jax import lax
from jax.experimental import pallas as pl
from jax.experimental.pallas import tpu as pltpu
from jax.experimental.pallas import tpu_sc as plsc

BN_EPS = 1e-5
_BNC = 1.0 / (1.0 + BN_EPS) ** 0.5

SC_CORES = 2
SC_SUBCORES = 16
SC_LANES = 16
SC_WORKERS = SC_CORES * SC_SUBCORES


def _bf16_bits(q):
    i = jax.lax.bitcast_convert_type(q, jnp.int32)
    rnd = jax.lax.shift_right_logical(i, 16) & 1
    return (i + 0x7FFF + rnd) & jnp.int32(-65536)


def _edge_q_body(ea0_ref, ea1_ref, w1_ref, b1_ref, w2_ref, b2_ref, we_ref,
                 be_ref, q0_ref, q1_ref, q2_ref, q3_ref):
    def enc(ea):
        t = jnp.maximum(
            jnp.dot(ea, w1_ref[...], preferred_element_type=jnp.float32)
            + b1_ref[...], 0.0)
        return (jnp.dot(t, w2_ref[...], preferred_element_type=jnp.float32)
                + b2_ref[...])

    e0 = enc(ea0_ref[...])
    e1 = enc(ea1_ref[...])
    for l, q_ref in enumerate((q0_ref, q1_ref, q2_ref, q3_ref)):
        qa = (jnp.dot(e0, we_ref[l], preferred_element_type=jnp.float32)
              + be_ref[l])
        qb = (jnp.dot(e1, we_ref[l], preferred_element_type=jnp.float32)
              + be_ref[l])
        lo = jax.lax.shift_right_logical(_bf16_bits(qa), 16)
        q_ref[...] = lo | _bf16_bits(qb)


def _edge_q(edge_attr, ee_W1, ee_b1, ee_W2, ee_b2, We, be, *, block_e):
    E, D = edge_attr.shape
    H = ee_W2.shape[1]
    hb = block_e // 2
    nblk = (E // 2) // hb
    out_sds = jax.ShapeDtypeStruct((E // 2, H), jnp.int32)
    row_spec = pl.BlockSpec((hb, H), lambda i: (i, 0))
    full = lambda s: pl.BlockSpec(s, lambda i: (0,) * len(s))
    return pl.pallas_call(
        _edge_q_body,
        grid=(nblk,),
        in_specs=[
            pl.BlockSpec((hb, D), lambda i: (i, 0)),
            pl.BlockSpec((hb, D), lambda i, _n=nblk: (i + _n, 0)),
            full(ee_W1.shape), full((1, H)), full(ee_W2.shape), full((1, H)),
            full(We.shape), full((4, 1, H)),
        ],
        out_specs=[row_spec] * 4,
        out_shape=[out_sds] * 4,
    )(edge_attr, edge_attr, ee_W1, ee_b1.reshape(1, H), ee_W2,
      ee_b2.reshape(1, H), We, be.reshape(4, 1, H))


def _sc_message_body(n_pad, n_edges, chunk, h_hbm, q_hbm, src_hbm, dst_hbm,
                     z_hbm, out_hbm, srcv, dstv, qv, rowsv, aggr,
                     sem_ld, sem_g, sem_sc):
    H = h_hbm.shape[1]
    c = lax.axis_index("c")
    s = lax.axis_index("s")
    wid = c * SC_SUBCORES + s
    per_w = n_edges // SC_WORKERS
    base = wid * per_w
    nchunks = per_w // chunk

    chunk2 = chunk // 2
    half = n_edges // 2
    base2 = wid * (per_w // 2)

    def issue_loads(b2, d4, ch):
        off = base2 + ch * chunk2
        pltpu.async_copy(src_hbm.at[pl.ds(off, chunk2)],
                         srcv.at[b2].at[pl.ds(0, chunk2)], sem_ld.at[b2])
        pltpu.async_copy(src_hbm.at[pl.ds(off + half, chunk2)],
                         srcv.at[b2].at[pl.ds(chunk2, chunk2)],
                         sem_ld.at[b2])
        pltpu.async_copy(dst_hbm.at[pl.ds(off, chunk2)],
                         dstv.at[d4].at[pl.ds(0, chunk2)], sem_ld.at[b2])
        pltpu.async_copy(dst_hbm.at[pl.ds(off + half, chunk2)],
                         dstv.at[d4].at[pl.ds(chunk2, chunk2)],
                         sem_ld.at[b2])
        pltpu.async_copy(q_hbm.at[pl.ds(off, chunk2)], qv.at[b2],
                         sem_ld.at[b2])

    def wait_loads(b2, d4):
        for _ in range(2):
            pltpu.make_async_copy(src_hbm.at[pl.ds(0, chunk2)],
                                  srcv.at[b2].at[pl.ds(0, chunk2)],
                                  sem_ld.at[b2]).wait()
            pltpu.make_async_copy(dst_hbm.at[pl.ds(0, chunk2)],
                                  dstv.at[d4].at[pl.ds(0, chunk2)],
                                  sem_ld.at[b2]).wait()
        pltpu.make_async_copy(q_hbm.at[pl.ds(0, chunk2)], qv.at[b2],
                              sem_ld.at[b2]).wait()

    def issue_gather(b2):
        pltpu.async_copy(h_hbm.at[srcv.at[b2]], rowsv.at[b2], sem_g.at[b2])

    def wait_gather(b2):
        pltpu.make_async_copy(h_hbm.at[srcv.at[b2]], rowsv.at[b2],
                              sem_g.at[b2]).wait()

    MASK_HI = jnp.int32(-65536)

    def compute(b2):
        rows = rowsv.at[b2]
        qb = qv.at[b2]

        @pl.loop(0, chunk2)
        def _(j):
            for g in range(H // SC_LANES):
                fsl = pl.ds(g * SC_LANES, SC_LANES)
                qw = qb.at[pl.ds(j, 1), fsl][...]
                q_lo = lax.bitcast_convert_type(qw << 16, jnp.float32)
                q_hi = lax.bitcast_convert_type(qw & MASK_HI, jnp.float32)
                sl_lo = (pl.ds(j, 1), fsl)
                sl_hi = (pl.ds(j + chunk2, 1), fsl)
                rows.at[*sl_lo][...] = jnp.maximum(
                    rows.at[*sl_lo][...] + q_lo, 0.0)
                rows.at[*sl_hi][...] = jnp.maximum(
                    rows.at[*sl_hi][...] + q_hi, 0.0)

    def issue_scatter(b2, d4):
        pltpu.async_copy(rowsv.at[b2], aggr.at[dstv.at[d4]], sem_sc.at[b2],
                         add=True)

    def wait_scatter(b2, d4):
        pltpu.make_async_copy(rowsv.at[b2], aggr.at[dstv.at[d4]],
                              sem_sc.at[b2]).wait()

    issue_loads(0, 0, 0)
    issue_loads(1, 1, 1)
    rows_per_sub = n_pad // SC_SUBCORES
    r0 = s * rows_per_sub
    pltpu.sync_copy(z_hbm.at[pl.ds(r0, rows_per_sub)],
                    aggr.at[pl.ds(r0, rows_per_sub)])
    plsc.subcore_barrier()

    wait_loads(0, 0)
    issue_gather(0)

    nmain = (nchunks - 1) // 4
    assert nmain * 4 == nchunks - 1, "nchunks must be 1 mod 4"

    @pl.loop(0, nmain)
    def _(i):
        for k in range(4):
            ch = i * 4 + k
            b2, o2 = k % 2, (k + 1) % 2
            wait_gather(b2)
            if k == 0:
                @pl.when(i >= 1)
                def _():
                    wait_scatter(o2, 3)
            else:
                wait_scatter(o2, k - 1)
            wait_loads(o2, (k + 1) % 4)
            issue_gather(o2)
            compute(b2)
            issue_scatter(b2, k)
            if k < 3:
                issue_loads(b2, (k + 2) % 4, ch + 2)
            else:
                @pl.when(ch + 2 < nchunks)
                def _():
                    issue_loads(b2, (k + 2) % 4, ch + 2)

    wait_gather(0)
    wait_scatter(1, 3)
    compute(0)
    issue_scatter(0, 0)
    wait_scatter(0, 0)

    plsc.subcore_barrier()
    pltpu.sync_copy(aggr.at[pl.ds(r0, rows_per_sub)],
                    out_hbm.at[c].at[pl.ds(r0, rows_per_sub)])


def _sc_message(h, q_w, src, dst, zeros_pad, *, chunk):
    N, H = h.shape
    E = src.shape[0]
    n_pad = zeros_pad.shape[0]
    mesh = plsc.VectorSubcoreMesh(core_axis_name="c", subcore_axis_name="s")
    body = functools.partial(_sc_message_body, n_pad, E, chunk)
    cp = pltpu.CompilerParams()
    k = pl.kernel(
        body,
        out_type=jax.ShapeDtypeStruct((SC_CORES, n_pad, H), jnp.float32),
        mesh=mesh,
        compiler_params=cp,
        scratch_types=[
            pltpu.VMEM((2, chunk), jnp.int32),
            pltpu.VMEM((4, chunk), jnp.int32),
            pltpu.VMEM((2, chunk // 2, H), jnp.int32),
            pltpu.VMEM((2, chunk, H), jnp.float32),
            pltpu.VMEM_SHARED((n_pad, H), jnp.float32),
            pltpu.SemaphoreType.DMA((2,)),
            pltpu.SemaphoreType.DMA((2,)),
            pltpu.SemaphoreType.DMA((2,)),
        ],
    )
    return k(h, q_w, src, dst, zeros_pad)


def _padded_zeros(N, H):
    quantum = SC_SUBCORES * 8
    n_pad = ((N + quantum - 1) // quantum) * quantum
    return jnp.zeros((n_pad, H), jnp.float32)


def _node_mlp_body(h_ref, p_ref, w1_ref, b1_ref, w2_ref, b2_ref, gb_ref,
                   o_ref):
    z = h_ref[...] + p_ref[0] + p_ref[1]
    z = jnp.maximum(
        jnp.dot(z, w1_ref[...], preferred_element_type=jnp.float32)
        + b1_ref[...], 0.0)
    z = (jnp.dot(z, w2_ref[...], preferred_element_type=jnp.float32)
         + b2_ref[...])
    z = jnp.maximum(z, 0.0)
    o_ref[...] = z * (_BNC * gb_ref[0]) + gb_ref[1]


def _node_mlp(h, partials, W1p, b1, W2p, b2p, gp, betap, *, block_n):
    N, H = h.shape
    nblk = N // block_n
    full = lambda s: pl.BlockSpec(s, lambda i: (0,) * len(s))
    row_spec = pl.BlockSpec((block_n, H), lambda i: (i, 0))
    return pl.pallas_call(
        _node_mlp_body,
        grid=(nblk,),
        in_specs=[
            row_spec,
            pl.BlockSpec((SC_CORES, block_n, H), lambda i: (0, i, 0)),
            full(W1p.shape), full((1, H)), full(W2p.shape), full((1, H)),
            full((2, 1, H)),
        ],
        out_specs=row_spec,
        out_shape=jax.ShapeDtypeStruct((N, H), jnp.float32),
    )(h, partials, W1p, b1.reshape(1, H), W2p, b2p.reshape(1, H),
      jnp.stack([gp, betap]).reshape(2, 1, H))


def _pool_head_body(h_ref, b3_ref, lat_ref, lw1_ref, lb1_ref, lgb_ref,
                    lw2_ref, lb2_ref, fw1_ref, fb1_ref, fgb_ref, fw2_ref,
                    fb2_ref, o_ref, sum_ref, cnt_ref, *, n_groups):
    i = pl.program_id(0)
    nblk = pl.num_programs(0)

    @pl.when(i == 0)
    def _():
        sum_ref[...] = jnp.zeros_like(sum_ref)
        cnt_ref[...] = jnp.zeros_like(cnt_ref)

    b = b3_ref[0]
    gids = lax.broadcasted_iota(jnp.int32, (n_groups, b.shape[1]), 0)
    onehot = (b == gids).astype(jnp.float32)
    sum_ref[...] += jnp.dot(onehot, h_ref[...],
                            preferred_element_type=jnp.float32)
    cnt_ref[...] += jnp.sum(onehot, axis=1, keepdims=True)

    @pl.when(i == nblk - 1)
    def _():
        pooled = sum_ref[...] / jnp.maximum(cnt_ref[...], 1.0)
        lf = jnp.maximum(
            jnp.dot(lat_ref[...], lw1_ref[...],
                    preferred_element_type=jnp.float32) + lb1_ref[...], 0.0)
        lf = lf * (_BNC * lgb_ref[0]) + lgb_ref[1]
        lf = (jnp.dot(lf, lw2_ref[...], preferred_element_type=jnp.float32)
              + lb2_ref[...])
        H = pooled.shape[1]
        y = (jnp.dot(pooled, fw1_ref[pl.ds(0, H)],
                     preferred_element_type=jnp.float32)
             + jnp.dot(lf, fw1_ref[pl.ds(H, H)],
                       preferred_element_type=jnp.float32)
             + fb1_ref[...])
        y = jnp.maximum(y, 0.0)
        y = y * (_BNC * fgb_ref[0]) + fgb_ref[1]
        o_ref[...] = (jnp.dot(y, fw2_ref[...],
                              preferred_element_type=jnp.float32)
                      + fb2_ref[...])


def _pool_head(h, batch, lattice, p, *, block_n):
    N, H = h.shape
    G = lattice.shape[0]
    NC_OUT = p['f_W2'].shape[1]
    nblk = N // block_n
    batch3 = batch.reshape(nblk, 1, block_n)
    full = lambda s: pl.BlockSpec(s, lambda i: (0,) * len(s))
    body = functools.partial(_pool_head_body, n_groups=G)
    return pl.pallas_call(
        body,
        grid=(nblk,),
        in_specs=[
            pl.BlockSpec((block_n, H), lambda i: (i, 0)),
            pl.BlockSpec((1, 1, block_n), lambda i: (i, 0, 0)),
            full(lattice.shape), full(p['lat_W1'].shape), full((1, H)),
            full((2, 1, H)), full(p['lat_W2'].shape), full((1, H)),
            full(p['f_W1'].shape), full((1, H)), full((2, 1, H)),
            full(p['f_W2'].shape), full((1, NC_OUT)),
        ],
        out_specs=full((G, NC_OUT)),
        out_shape=jax.ShapeDtypeStruct((G, NC_OUT), jnp.float32),
        scratch_shapes=[
            pltpu.VMEM((G, H), jnp.float32),
            pltpu.VMEM((G, 1), jnp.float32),
        ],
    )(h, batch3, lattice, p['lat_W1'], p['lat_b1'].reshape(1, H),
      jnp.stack([p['lat_g'], p['lat_beta']]).reshape(2, 1, H),
      p['lat_W2'], p['lat_b2'].reshape(1, H),
      p['f_W1'], p['f_b1'].reshape(1, H),
      jnp.stack([p['f_g'], p['f_beta']]).reshape(2, 1, H),
      p['f_W2'], p['f_b2'].reshape(1, NC_OUT))


def kernel(x, edge_attr, lattice, params, edge_index, batch):
    p = params
    N, H = x.shape
    src = edge_index[0]
    dst = edge_index[1]

    We = jnp.stack([lp['We'] for lp in p['gnn']])
    be = jnp.stack([lp['be'] for lp in p['gnn']])

    qws = _edge_q(edge_attr, p['ee_W1'], p['ee_b1'], p['ee_W2'], p['ee_b2'],
                  We, be, block_e=1600)

    zeros_pad = _padded_zeros(N, H)
    h = x
    for l, lp in enumerate(p['gnn']):
        partials = _sc_message(h, qws[l], src, dst, zeros_pad, chunk=80)
        h = _node_mlp(h, partials, lp['W1'], lp['b1'], lp['W2'], lp['b2'],
                      lp['g'], lp['beta'], block_n=1000)

    return _pool_head(h, batch, lattice, p, block_n=1000)
